# baseline (speedup 1.0000x reference)
import functools

import jax
import jax.numpy as jnp
from jax import lax
from jax.experimental import pallas as pl
from jax.experimental.pallas import tpu as pltpu
from jax.experimental.pallas import tpu_sc as plsc

L = 16
TILE = 256
GCH = 128


def _index_kernel_body(t_len, n_env, masks_hbm, perm_hbm, meta_hbm,
                       h0lane_hbm, m_v, rel_v, eid_v, pos_v, len_v, lane_v,
                       hist_v, b_v, off_v, occ_v, meta_v, row_v, hist_sh):
    cid = lax.axis_index("c")
    sid = lax.axis_index("s")
    nchunks = t_len // L
    tbl_ch = (t_len + 2 * L) // L

    @pl.when(cid == 0)
    def _worker():
        env = sid
        pltpu.sync_copy(masks_hbm.at[pl.ds(env * t_len, t_len)], m_v)

        zeros = jnp.zeros((L,), jnp.int32)
        def zero_body(i, _):
            hist_v[pl.ds(i * L, L)] = zeros
            occ_v[pl.ds(i * L, L)] = zeros
            return 0
        lax.fori_loop(0, tbl_ch, zero_body, 0)

        iota = lax.iota(jnp.int32, L)

        def p1(i, carry):
            eid_c, ls_c = carry
            tv = iota + i * L
            m = m_v[pl.ds(i * L, L)]
            st = jnp.where((tv == 0) | (m == 0), 1, 0).astype(jnp.int32)
            eidv = plsc.cumsum(st) + eid_c
            lsv = jnp.maximum(plsc.cummax(jnp.where(st == 1, tv, -1)), ls_c)
            rel_v[pl.ds(i * L, L)] = tv - lsv
            eid_v[pl.ds(i * L, L)] = eidv
            plsc.store_scatter(pos_v, [eidv], tv, mask=st == 1)
            return (jnp.max(eidv), jnp.max(lsv))
        e_cnt, _ = lax.fori_loop(0, nchunks, p1, (jnp.int32(0), jnp.int32(-1)))

        def p2(i, _):
            ev = iota + 1 + i * L
            valid = ev <= e_cnt
            p_here = plsc.load_gather(pos_v, [jnp.where(valid, ev, 0)])
            nxt = plsc.load_gather(pos_v, [jnp.where(ev < e_cnt, ev + 1, 0)])
            lenv = jnp.where(ev == e_cnt, t_len - p_here, nxt - p_here)
            lenv = jnp.where(valid, lenv, 0)
            plsc.store_scatter(len_v, [ev], lenv, mask=valid)
            plsc.addupdate_scatter(hist_v, [lenv],
                                   jnp.ones((L,), jnp.int32), mask=valid)
            return 0
        lax.fori_loop(0, nchunks, p2, 0)

        pltpu.sync_copy(hist_v, hist_sh.at[env])
        plsc.subcore_barrier()

        def add_env(e, _):
            pltpu.sync_copy(hist_sh.at[e], meta_v)
            def addc(i, _):
                hv = meta_v[pl.ds(i * L, L)]
                hist_v[pl.ds(i * L, L)] = hist_v[pl.ds(i * L, L)] + hv
                occ_v[pl.ds(i * L, L)] = occ_v[pl.ds(i * L, L)] + jnp.where(
                    e < env, hv, 0)
                return 0
            lax.fori_loop(0, tbl_ch, addc, 0)
            return 0
        def zero_hist(i, _):
            hist_v[pl.ds(i * L, L)] = zeros
            return 0
        lax.fori_loop(0, tbl_ch, zero_hist, 0)
        lax.fori_loop(0, n_env, add_env, 0)

        def p3(i, carry):
            tot, mx_c = carry
            hv = hist_v[pl.ds(i * L, L)]
            pref = plsc.cumsum(hv) + tot
            b_v[pl.ds(i * L, L)] = pref
            sv = iota + i * L
            mx = jnp.max(jnp.where(hv > 0, sv, 0))
            return (jnp.max(pref), jnp.maximum(mx_c, mx))
        carry = lax.fori_loop(0, tbl_ch, p3, (jnp.int32(0), jnp.int32(0)))
        e_tot, s_max = carry
        def p3b(i, _):
            b_v[pl.ds(i * L, L)] = e_tot - b_v[pl.ds(i * L, L)]
            return 0
        lax.fori_loop(0, tbl_ch, p3b, 0)

        def p3c(i, off_c):
            bv = b_v[pl.ds(i * L, L)]
            rb = (bv + 7) // 8 * 8
            cs = plsc.cumsum(rb)
            off_v[pl.ds(i * L, L)] = cs - rb + off_c
            return off_c + cs[L - 1]
        lax.fori_loop(0, tbl_ch, p3c, jnp.int32(0))

        lane0 = iota == 0
        def p4(e, _):
            e_vec = jnp.zeros((L,), jnp.int32) + e
            ln_vec = plsc.load_gather(len_v, [e_vec])
            b_l = plsc.load_gather(b_v, [ln_vec])
            occ_l = plsc.load_gather(occ_v, [ln_vec])
            plsc.store_scatter(lane_v, [e_vec], b_l + occ_l, mask=lane0)
            plsc.store_scatter(occ_v, [ln_vec], occ_l + 1, mask=lane0)
            return 0
        lax.fori_loop(1, e_cnt + 1, p4, 0)

        def p5(i, _):
            relv = rel_v[pl.ds(i * L, L)]
            eidv = eid_v[pl.ds(i * L, L)]
            lanes = plsc.load_gather(lane_v, [eidv])
            offs = plsc.load_gather(off_v, [relv])
            row_v[pl.ds(i * L, L)] = offs + lanes
            return 0
        lax.fori_loop(0, nchunks, p5, 0)
        pltpu.sync_copy(row_v, perm_hbm.at[pl.ds(env * t_len, t_len)])

        pltpu.sync_copy(lane_v.at[pl.ds(0, L)], h0lane_hbm.at[pl.ds(env * L, L)])

        @pl.when(env == 0)
        def _meta():
            meta_v[pl.ds(0, L)] = jnp.where(iota == 0, s_max, 0)
            def mcopy(i, _):
                ov = off_v[pl.ds(i * L, L)]
                sv = iota + i * L
                plsc.store_scatter(meta_v, [sv + 1], ov,
                                   mask=sv <= t_len)
                return 0
            lax.fori_loop(0, tbl_ch, mcopy, 0)
            pltpu.sync_copy(meta_v, meta_hbm)


def _build_index(masks_nt, n_env):
    t_len = masks_nt.shape[0] // n_env
    r_tot = t_len * n_env
    tbl = ((t_len + 2 * L) // L) * L
    mesh = plsc.VectorSubcoreMesh(core_axis_name="c", subcore_axis_name="s")

    body = functools.partial(_index_kernel_body, t_len, n_env)
    kern = functools.partial(
        pl.kernel, mesh=mesh,
        compiler_params=pltpu.CompilerParams(needs_layout_passes=False),
        out_type=(
            jax.ShapeDtypeStruct((r_tot,), jnp.int32),
            jax.ShapeDtypeStruct((tbl,), jnp.int32),
            jax.ShapeDtypeStruct((n_env * L,), jnp.int32),
        ),
        scratch_types=[
            pltpu.VMEM((t_len,), jnp.int32),
            pltpu.VMEM((t_len,), jnp.int32),
            pltpu.VMEM((t_len,), jnp.int32),
            pltpu.VMEM((tbl,), jnp.int32),
            pltpu.VMEM((tbl,), jnp.int32),
            pltpu.VMEM((tbl,), jnp.int32),
            pltpu.VMEM((tbl,), jnp.int32),
            pltpu.VMEM((tbl,), jnp.int32),
            pltpu.VMEM((tbl,), jnp.int32),
            pltpu.VMEM((tbl,), jnp.int32),
            pltpu.VMEM((tbl,), jnp.int32),
            pltpu.VMEM((t_len,), jnp.int32),
            pltpu.VMEM_SHARED((n_env, tbl), jnp.int32),
        ],
    )(body)
    return kern(masks_nt)


def _scatter_rows(rows_in, idx, out_rows):
    n_idx = idx.shape[0]
    d = rows_in.shape[1]
    mesh = plsc.VectorSubcoreMesh(core_axis_name="c", subcore_axis_name="s")
    info = plsc.get_sparse_core_info()
    nw = info.num_cores * info.num_subcores
    per_w = n_idx // nw
    n_ch = per_w // GCH

    def body(rows_hbm, idx_hbm, out_hbm, idx2_v, rows_v, sem):
        wid = lax.axis_index("s") * info.num_cores + lax.axis_index("c")
        base = wid * per_w
        def ld_idx(j, _):
            pltpu.sync_copy(idx_hbm.at[pl.ds(base + j * GCH, GCH)],
                            idx2_v.at[j])
            return 0
        lax.fori_loop(0, n_ch, ld_idx, 0)
        def chunk(j, _):
            pltpu.sync_copy(rows_hbm.at[pl.ds(base + j * GCH, GCH)], rows_v)
            pltpu.async_copy(rows_v, out_hbm.at[idx2_v.at[j]], sem).wait()
            return 0
        lax.fori_loop(0, n_ch, chunk, 0)

    kern = functools.partial(
        pl.kernel, mesh=mesh,
        compiler_params=pltpu.CompilerParams(needs_layout_passes=False),
        out_type=jax.ShapeDtypeStruct((out_rows, d), jnp.float32),
        scratch_types=[
            pltpu.VMEM((per_w // GCH, GCH), jnp.int32),
            pltpu.VMEM((GCH, d), jnp.float32),
            pltpu.SemaphoreType.DMA,
        ],
    )(body)
    return kern(rows_in, idx)


def _gather_rows(table, idx, out_rows):
    n_idx = idx.shape[0]
    d = table.shape[1]
    mesh = plsc.VectorSubcoreMesh(core_axis_name="c", subcore_axis_name="s")
    info = plsc.get_sparse_core_info()
    nw = info.num_cores * info.num_subcores
    per_w = n_idx // nw

    def body(table_hbm, idx_hbm, out_hbm, idx_v, rows_v, sem):
        wid = lax.axis_index("s") * info.num_cores + lax.axis_index("c")
        base = wid * per_w
        def chunk(i, _):
            b = base + i * GCH
            pltpu.sync_copy(idx_hbm.at[pl.ds(b, GCH)], idx_v)
            pltpu.async_copy(table_hbm.at[idx_v], rows_v, sem).wait()
            pltpu.sync_copy(rows_v, out_hbm.at[pl.ds(b, GCH)])
            return 0
        lax.fori_loop(0, per_w // GCH, chunk, 0)

    kern = functools.partial(
        pl.kernel, mesh=mesh,
        compiler_params=pltpu.CompilerParams(needs_layout_passes=False),
        out_type=jax.ShapeDtypeStruct((out_rows, d), jnp.float32),
        scratch_types=[
            pltpu.VMEM((GCH,), jnp.int32),
            pltpu.VMEM((GCH, d), jnp.float32),
            pltpu.SemaphoreType.DMA,
        ],
    )(body)
    return kern(table, idx)


def _scan_body(n_env, h_dim, px_hbm, wih_ref, whh_ref, bih_ref,
               bhh_ref, h0_ref, m0_ref, meta_ref, lanes_ref, pout_hbm,
               xa_ref, xb_ref, ha_ref, hb_ref, oa_ref, ob_ref,
               sx0, sx1, sh0, sh1, so0, so1):
    s_max = meta_ref[0]
    wih = wih_ref[...].astype(jnp.bfloat16)
    whh = whh_ref[...].astype(jnp.bfloat16)
    bih = bih_ref[...]
    bhh = bhh_ref[...]

    def xcp(base, buf, sem):
        return pltpu.make_async_copy(
            px_hbm.at[pl.ds(pl.multiple_of(base, 8), TILE)], buf, sem)

    def hcp(base, buf, sem):
        return pltpu.make_async_copy(
            pout_hbm.at[pl.ds(pl.multiple_of(base, 8), TILE)], buf, sem)

    def ocp(buf, base, sem):
        return pltpu.make_async_copy(
            buf, pout_hbm.at[pl.ds(pl.multiple_of(base, 8), TILE)], sem)

    def step(s, _):
        off_s = meta_ref[1 + s]
        b_s = meta_ref[2 + s] - off_s
        off_p = meta_ref[s]
        nb = (b_s + TILE - 1) // TILE

        xcp(off_s, xa_ref, sx0).start()
        @pl.when(s > 0)
        def _():
            hcp(off_p, ha_ref, sh0).start()

        def tile(tb, _):
            base = pl.multiple_of(off_s + tb * TILE, 8)
            even = tb % 2 == 0

            @pl.when(tb + 1 < nb)
            def _prefetch():
                nx = off_s + (tb + 1) * TILE
                nh = off_p + (tb + 1) * TILE
                @pl.when(even)
                def _():
                    xcp(nx, xb_ref, sx1).start()
                    @pl.when(s > 0)
                    def _():
                        hcp(nh, hb_ref, sh1).start()
                @pl.when(jnp.logical_not(even))
                def _():
                    xcp(nx, xa_ref, sx0).start()
                    @pl.when(s > 0)
                    def _():
                        hcp(nh, ha_ref, sh0).start()

            def proc(xr, hr, orr, sx, sh, so):
                @pl.when(s > 0)
                def _():
                    hcp(base, hr, sh).wait()
                @pl.when(s == 0)
                def _init():
                    hr[...] = jnp.zeros((TILE, h_dim), jnp.float32)
                    for k in range(n_env):
                        lane = lanes_ref[k]
                        @pl.when((lane >= tb * TILE) &
                                 (lane < tb * TILE + TILE))
                        def _():
                            hr[pl.ds(lane - tb * TILE, 1), :] = (
                                h0_ref[k:k + 1, :] * m0_ref[k:k + 1, 0:1])
                xcp(base, xr, sx).wait()
                gi = jnp.dot(xr[...].astype(jnp.bfloat16), wih,
                             preferred_element_type=jnp.float32) + bih
                hp = hr[...]
                gh = jnp.dot(hp.astype(jnp.bfloat16), whh,
                             preferred_element_type=jnp.float32) + bhh
                r = jax.nn.sigmoid(gi[:, :h_dim] + gh[:, :h_dim])
                z = jax.nn.sigmoid(
                    gi[:, h_dim:2 * h_dim] + gh[:, h_dim:2 * h_dim])
                ng = jnp.tanh(gi[:, 2 * h_dim:] + r * gh[:, 2 * h_dim:])
                @pl.when(tb >= 2)
                def _():
                    ocp(orr, base, so).wait()
                orr[...] = ng + z * (hp - ng)
                ocp(orr, base, so).start()

            @pl.when(even)
            def _():
                proc(xa_ref, ha_ref, oa_ref, sx0, sh0, so0)
            @pl.when(jnp.logical_not(even))
            def _():
                proc(xb_ref, hb_ref, ob_ref, sx1, sh1, so1)
            return 0

        lax.fori_loop(0, nb, tile, 0)

        @pl.when(nb % 2 == 1)
        def _():
            ocp(oa_ref, off_s, so0).wait()
            @pl.when(nb >= 2)
            def _():
                ocp(ob_ref, off_s, so1).wait()
        @pl.when(nb % 2 == 0)
        def _():
            ocp(ob_ref, off_s, so1).wait()
            @pl.when(nb >= 2)
            def _():
                ocp(oa_ref, off_s, so0).wait()
        return 0

    lax.fori_loop(0, s_max, step, 0)


def _packed_scan(px, wih_t, whh_t, bih2, bhh2, h0, m0col, meta, lanes):
    rows = px.shape[0]
    h_dim = whh_t.shape[0]
    n_env = h0.shape[0]
    body = functools.partial(_scan_body, n_env, h_dim)
    return pl.pallas_call(
        body,
        in_specs=[
            pl.BlockSpec(memory_space=pltpu.HBM),
            pl.BlockSpec(memory_space=pltpu.VMEM),
            pl.BlockSpec(memory_space=pltpu.VMEM),
            pl.BlockSpec(memory_space=pltpu.VMEM),
            pl.BlockSpec(memory_space=pltpu.VMEM),
            pl.BlockSpec(memory_space=pltpu.VMEM),
            pl.BlockSpec(memory_space=pltpu.VMEM),
            pl.BlockSpec(memory_space=pltpu.SMEM),
            pl.BlockSpec(memory_space=pltpu.SMEM),
        ],
        out_specs=pl.BlockSpec(memory_space=pltpu.HBM),
        out_shape=jax.ShapeDtypeStruct((rows, h_dim), jnp.float32),
        scratch_shapes=[
            pltpu.VMEM((TILE, px.shape[1]), jnp.float32),
            pltpu.VMEM((TILE, px.shape[1]), jnp.float32),
            pltpu.VMEM((TILE, h_dim), jnp.float32),
            pltpu.VMEM((TILE, h_dim), jnp.float32),
            pltpu.VMEM((TILE, h_dim), jnp.float32),
            pltpu.VMEM((TILE, h_dim), jnp.float32),
            pltpu.SemaphoreType.DMA,
            pltpu.SemaphoreType.DMA,
            pltpu.SemaphoreType.DMA,
            pltpu.SemaphoreType.DMA,
            pltpu.SemaphoreType.DMA,
            pltpu.SemaphoreType.DMA,
        ],
    )(px, wih_t, whh_t, bih2, bhh2, h0, m0col, meta, lanes)


def kernel(x, hidden_states, masks, W_ih, W_hh, b_ih, b_hh):
    n = hidden_states.shape[1]
    h_dim = hidden_states.shape[2]
    t = x.shape[0] // n
    r_tot = t * n

    ms = masks.reshape(t, n)
    masks_nt = ms.T.astype(jnp.int32).reshape(r_tot)
    perm_nt, meta, h0lanes = _build_index(masks_nt, n)
    perm_r = perm_nt.reshape(n, t).T.reshape(r_tot)
    lanes16 = h0lanes.reshape(n, L)[:, 1]

    r_pad = r_tot + 8 * t
    packed_x = _scatter_rows(x, perm_r, r_pad + TILE)

    wih_t = W_ih.T
    whh_t = W_hh.T
    bih2 = b_ih.reshape(1, 3 * h_dim)
    bhh2 = b_hh.reshape(1, 3 * h_dim)
    h0 = hidden_states[0]
    m0col = jnp.broadcast_to(
        ms[0].astype(jnp.float32).reshape(n, 1), (n, 128))

    packed_out = _packed_scan(packed_x, wih_t, whh_t, bih2, bhh2, h0,
                              m0col, meta, lanes16)

    out = _gather_rows(packed_out, perm_r, r_tot)
    h_final = out.reshape(t, n, h_dim)[-1][None]
    return out, h_final


# speedup vs baseline: 1.0076x; 1.0076x over previous
import functools

import jax
import jax.numpy as jnp
from jax import lax
from jax.experimental import pallas as pl
from jax.experimental.pallas import tpu as pltpu
from jax.experimental.pallas import tpu_sc as plsc

L = 16
TILE = 256
GCH = 128


def _index_kernel_body(t_len, n_env, masks_hbm, perm_hbm, meta_hbm,
                       h0lane_hbm, m_v, rel_v, eid_v, pos_v, len_v, lane_v,
                       hist_v, b_v, off_v, occ_v, meta_v, row_v, hist_sh):
    cid = lax.axis_index("c")
    sid = lax.axis_index("s")
    nchunks = t_len // L
    tbl_ch = (t_len + 2 * L) // L

    @pl.when(cid == 0)
    def _worker():
        env = sid
        pltpu.sync_copy(masks_hbm.at[pl.ds(env * t_len, t_len)], m_v)

        zeros = jnp.zeros((L,), jnp.int32)
        def zero_body(i, _):
            hist_v[pl.ds(i * L, L)] = zeros
            occ_v[pl.ds(i * L, L)] = zeros
            return 0
        lax.fori_loop(0, tbl_ch, zero_body, 0)

        iota = lax.iota(jnp.int32, L)

        def p1(i, carry):
            eid_c, ls_c = carry
            tv = iota + i * L
            m = m_v[pl.ds(i * L, L)]
            st = jnp.where((tv == 0) | (m == 0), 1, 0).astype(jnp.int32)
            eidv = plsc.cumsum(st) + eid_c
            lsv = jnp.maximum(plsc.cummax(jnp.where(st == 1, tv, -1)), ls_c)
            rel_v[pl.ds(i * L, L)] = tv - lsv
            eid_v[pl.ds(i * L, L)] = eidv
            plsc.store_scatter(pos_v, [eidv], tv, mask=st == 1)
            return (jnp.max(eidv), jnp.max(lsv))
        e_cnt, _ = lax.fori_loop(0, nchunks, p1, (jnp.int32(0), jnp.int32(-1)))

        def p2(i, _):
            ev = iota + 1 + i * L
            valid = ev <= e_cnt
            p_here = plsc.load_gather(pos_v, [jnp.where(valid, ev, 0)])
            nxt = plsc.load_gather(pos_v, [jnp.where(ev < e_cnt, ev + 1, 0)])
            lenv = jnp.where(ev == e_cnt, t_len - p_here, nxt - p_here)
            lenv = jnp.where(valid, lenv, 0)
            plsc.store_scatter(len_v, [ev], lenv, mask=valid)
            plsc.addupdate_scatter(hist_v, [lenv],
                                   jnp.ones((L,), jnp.int32), mask=valid)
            return 0
        lax.fori_loop(0, nchunks, p2, 0)

        pltpu.sync_copy(hist_v, hist_sh.at[env])
        plsc.subcore_barrier()

        def add_env(e, _):
            pltpu.sync_copy(hist_sh.at[e], meta_v)
            def addc(i, _):
                hv = meta_v[pl.ds(i * L, L)]
                hist_v[pl.ds(i * L, L)] = hist_v[pl.ds(i * L, L)] + hv
                occ_v[pl.ds(i * L, L)] = occ_v[pl.ds(i * L, L)] + jnp.where(
                    e < env, hv, 0)
                return 0
            lax.fori_loop(0, tbl_ch, addc, 0)
            return 0
        def zero_hist(i, _):
            hist_v[pl.ds(i * L, L)] = zeros
            return 0
        lax.fori_loop(0, tbl_ch, zero_hist, 0)
        lax.fori_loop(0, n_env, add_env, 0)

        def p3(i, carry):
            tot, mx_c = carry
            hv = hist_v[pl.ds(i * L, L)]
            pref = plsc.cumsum(hv) + tot
            b_v[pl.ds(i * L, L)] = pref
            sv = iota + i * L
            mx = jnp.max(jnp.where(hv > 0, sv, 0))
            return (jnp.max(pref), jnp.maximum(mx_c, mx))
        carry = lax.fori_loop(0, tbl_ch, p3, (jnp.int32(0), jnp.int32(0)))
        e_tot, s_max = carry
        def p3b(i, _):
            b_v[pl.ds(i * L, L)] = e_tot - b_v[pl.ds(i * L, L)]
            return 0
        lax.fori_loop(0, tbl_ch, p3b, 0)

        def p3c(i, off_c):
            bv = b_v[pl.ds(i * L, L)]
            rb = (bv + 7) // 8 * 8
            cs = plsc.cumsum(rb)
            off_v[pl.ds(i * L, L)] = cs - rb + off_c
            return off_c + cs[L - 1]
        lax.fori_loop(0, tbl_ch, p3c, jnp.int32(0))

        lane0 = iota == 0
        def p4(e, _):
            e_vec = jnp.zeros((L,), jnp.int32) + e
            ln_vec = plsc.load_gather(len_v, [e_vec])
            b_l = plsc.load_gather(b_v, [ln_vec])
            occ_l = plsc.load_gather(occ_v, [ln_vec])
            plsc.store_scatter(lane_v, [e_vec], b_l + occ_l, mask=lane0)
            plsc.store_scatter(occ_v, [ln_vec], occ_l + 1, mask=lane0)
            return 0
        lax.fori_loop(1, e_cnt + 1, p4, 0)

        def p5(i, _):
            relv = rel_v[pl.ds(i * L, L)]
            eidv = eid_v[pl.ds(i * L, L)]
            lanes = plsc.load_gather(lane_v, [eidv])
            offs = plsc.load_gather(off_v, [relv])
            row_v[pl.ds(i * L, L)] = offs + lanes
            return 0
        lax.fori_loop(0, nchunks, p5, 0)
        pltpu.sync_copy(row_v, perm_hbm.at[pl.ds(env * t_len, t_len)])

        pltpu.sync_copy(lane_v.at[pl.ds(0, L)], h0lane_hbm.at[pl.ds(env * L, L)])

        @pl.when(env == 0)
        def _meta():
            meta_v[pl.ds(0, L)] = jnp.where(iota == 0, s_max, 0)
            def mcopy(i, _):
                ov = off_v[pl.ds(i * L, L)]
                sv = iota + i * L
                plsc.store_scatter(meta_v, [sv + 1], ov,
                                   mask=sv <= t_len)
                return 0
            lax.fori_loop(0, tbl_ch, mcopy, 0)
            pltpu.sync_copy(meta_v, meta_hbm)


def _build_index(masks_nt, n_env):
    t_len = masks_nt.shape[0] // n_env
    r_tot = t_len * n_env
    tbl = ((t_len + 2 * L) // L) * L
    mesh = plsc.VectorSubcoreMesh(core_axis_name="c", subcore_axis_name="s")

    body = functools.partial(_index_kernel_body, t_len, n_env)
    kern = functools.partial(
        pl.kernel, mesh=mesh,
        compiler_params=pltpu.CompilerParams(needs_layout_passes=False),
        out_type=(
            jax.ShapeDtypeStruct((r_tot,), jnp.int32),
            jax.ShapeDtypeStruct((tbl,), jnp.int32),
            jax.ShapeDtypeStruct((n_env * L,), jnp.int32),
        ),
        scratch_types=[
            pltpu.VMEM((t_len,), jnp.int32),
            pltpu.VMEM((t_len,), jnp.int32),
            pltpu.VMEM((t_len,), jnp.int32),
            pltpu.VMEM((tbl,), jnp.int32),
            pltpu.VMEM((tbl,), jnp.int32),
            pltpu.VMEM((tbl,), jnp.int32),
            pltpu.VMEM((tbl,), jnp.int32),
            pltpu.VMEM((tbl,), jnp.int32),
            pltpu.VMEM((tbl,), jnp.int32),
            pltpu.VMEM((tbl,), jnp.int32),
            pltpu.VMEM((tbl,), jnp.int32),
            pltpu.VMEM((t_len,), jnp.int32),
            pltpu.VMEM_SHARED((n_env, tbl), jnp.int32),
        ],
    )(body)
    return kern(masks_nt)


def _scatter_rows(rows_in, idx, out_rows):
    n_idx = idx.shape[0]
    d = rows_in.shape[1]
    mesh = plsc.VectorSubcoreMesh(core_axis_name="c", subcore_axis_name="s")
    info = plsc.get_sparse_core_info()
    nw = info.num_cores * info.num_subcores
    per_w = n_idx // nw
    n_ch = per_w // GCH

    def body(rows_hbm, idx_hbm, out_hbm, idx2_v, rows_v, sem):
        wid = lax.axis_index("s") * info.num_cores + lax.axis_index("c")
        base = wid * per_w
        def ld_idx(j, _):
            pltpu.sync_copy(idx_hbm.at[pl.ds(base + j * GCH, GCH)],
                            idx2_v.at[j])
            return 0
        lax.fori_loop(0, n_ch, ld_idx, 0)
        def chunk(j, _):
            pltpu.sync_copy(rows_hbm.at[pl.ds(base + j * GCH, GCH)], rows_v)
            pltpu.async_copy(rows_v, out_hbm.at[idx2_v.at[j]], sem).wait()
            return 0
        lax.fori_loop(0, n_ch, chunk, 0)

    kern = functools.partial(
        pl.kernel, mesh=mesh,
        compiler_params=pltpu.CompilerParams(needs_layout_passes=False),
        out_type=jax.ShapeDtypeStruct((out_rows, d), jnp.float32),
        scratch_types=[
            pltpu.VMEM((per_w // GCH, GCH), jnp.int32),
            pltpu.VMEM((GCH, d), jnp.float32),
            pltpu.SemaphoreType.DMA,
        ],
    )(body)
    return kern(rows_in, idx)


def _gather_rows(table, idx, out_rows):
    n_idx = idx.shape[0]
    d = table.shape[1]
    mesh = plsc.VectorSubcoreMesh(core_axis_name="c", subcore_axis_name="s")
    info = plsc.get_sparse_core_info()
    nw = info.num_cores * info.num_subcores
    per_w = n_idx // nw

    def body(table_hbm, idx_hbm, out_hbm, idx_v, rows_v, sem):
        wid = lax.axis_index("s") * info.num_cores + lax.axis_index("c")
        base = wid * per_w
        def chunk(i, _):
            b = base + i * GCH
            pltpu.sync_copy(idx_hbm.at[pl.ds(b, GCH)], idx_v)
            pltpu.async_copy(table_hbm.at[idx_v], rows_v, sem).wait()
            pltpu.sync_copy(rows_v, out_hbm.at[pl.ds(b, GCH)])
            return 0
        lax.fori_loop(0, per_w // GCH, chunk, 0)

    kern = functools.partial(
        pl.kernel, mesh=mesh,
        compiler_params=pltpu.CompilerParams(needs_layout_passes=False),
        out_type=jax.ShapeDtypeStruct((out_rows, d), jnp.float32),
        scratch_types=[
            pltpu.VMEM((GCH,), jnp.int32),
            pltpu.VMEM((GCH, d), jnp.float32),
            pltpu.SemaphoreType.DMA,
        ],
    )(body)
    return kern(table, idx)


def _scan_body(n_env, h_dim, px_hbm, wih_ref, whh_ref, bih_ref,
               bhh_ref, h0_ref, m0_ref, meta_ref, lanes_ref, pout_hbm,
               xa_ref, xb_ref, ha_ref, hb_ref, oa_ref, ob_ref,
               sx0, sx1, sh0, sh1, so0, so1):
    s_max = meta_ref[0]
    wih = wih_ref[...].astype(jnp.bfloat16)
    whh = whh_ref[...].astype(jnp.bfloat16)
    bih = bih_ref[...]
    bhh = bhh_ref[...]

    def xcp(base, buf, sem):
        return pltpu.make_async_copy(
            px_hbm.at[pl.ds(pl.multiple_of(base, 8), TILE)], buf, sem)

    def hcp(base, buf, sem):
        return pltpu.make_async_copy(
            pout_hbm.at[pl.ds(pl.multiple_of(base, 8), TILE)], buf, sem)

    def ocp(buf, base, sem):
        return pltpu.make_async_copy(
            buf, pout_hbm.at[pl.ds(pl.multiple_of(base, 8), TILE)], sem)

    def step(s, _):
        off_s = meta_ref[1 + s]
        b_s = meta_ref[2 + s] - off_s
        off_p = meta_ref[s]
        nb = (b_s + TILE - 1) // TILE

        xcp(off_s, xa_ref, sx0).start()
        @pl.when(s > 0)
        def _():
            hcp(off_p, ha_ref, sh0).start()

        def tile(tb, _):
            base = pl.multiple_of(off_s + tb * TILE, 8)
            even = tb % 2 == 0

            @pl.when(tb + 1 < nb)
            def _prefetch():
                nx = off_s + (tb + 1) * TILE
                nh = off_p + (tb + 1) * TILE
                @pl.when(even)
                def _():
                    xcp(nx, xb_ref, sx1).start()
                    @pl.when(s > 0)
                    def _():
                        hcp(nh, hb_ref, sh1).start()
                @pl.when(jnp.logical_not(even))
                def _():
                    xcp(nx, xa_ref, sx0).start()
                    @pl.when(s > 0)
                    def _():
                        hcp(nh, ha_ref, sh0).start()

            def proc(xr, hr, orr, sx, sh, so):
                @pl.when(s > 0)
                def _():
                    hcp(base, hr, sh).wait()
                @pl.when(s == 0)
                def _init():
                    hr[...] = jnp.zeros((TILE, h_dim), jnp.float32)
                    for k in range(n_env):
                        lane = lanes_ref[k]
                        @pl.when((lane >= tb * TILE) &
                                 (lane < tb * TILE + TILE))
                        def _():
                            hr[pl.ds(lane - tb * TILE, 1), :] = (
                                h0_ref[k:k + 1, :] * m0_ref[k:k + 1, 0:1])
                xcp(base, xr, sx).wait()
                gi = jnp.dot(xr[...].astype(jnp.bfloat16), wih,
                             preferred_element_type=jnp.float32) + bih
                hp = hr[...]
                gh = jnp.dot(hp.astype(jnp.bfloat16), whh,
                             preferred_element_type=jnp.float32) + bhh
                r = jax.nn.sigmoid(gi[:, :h_dim] + gh[:, :h_dim])
                z = jax.nn.sigmoid(
                    gi[:, h_dim:2 * h_dim] + gh[:, h_dim:2 * h_dim])
                ng = jnp.tanh(gi[:, 2 * h_dim:] + r * gh[:, 2 * h_dim:])
                @pl.when(tb >= 2)
                def _():
                    ocp(orr, base, so).wait()
                orr[...] = (1.0 - z) * ng + z * hp
                ocp(orr, base, so).start()

            @pl.when(even)
            def _():
                proc(xa_ref, ha_ref, oa_ref, sx0, sh0, so0)
            @pl.when(jnp.logical_not(even))
            def _():
                proc(xb_ref, hb_ref, ob_ref, sx1, sh1, so1)
            return 0

        lax.fori_loop(0, nb, tile, 0)

        @pl.when(nb % 2 == 1)
        def _():
            ocp(oa_ref, off_s, so0).wait()
            @pl.when(nb >= 2)
            def _():
                ocp(ob_ref, off_s, so1).wait()
        @pl.when(nb % 2 == 0)
        def _():
            ocp(ob_ref, off_s, so1).wait()
            @pl.when(nb >= 2)
            def _():
                ocp(oa_ref, off_s, so0).wait()
        return 0

    lax.fori_loop(0, s_max, step, 0)


def _packed_scan(px, wih_t, whh_t, bih2, bhh2, h0, m0col, meta, lanes):
    rows = px.shape[0]
    h_dim = whh_t.shape[0]
    n_env = h0.shape[0]
    body = functools.partial(_scan_body, n_env, h_dim)
    return pl.pallas_call(
        body,
        in_specs=[
            pl.BlockSpec(memory_space=pltpu.HBM),
            pl.BlockSpec(memory_space=pltpu.VMEM),
            pl.BlockSpec(memory_space=pltpu.VMEM),
            pl.BlockSpec(memory_space=pltpu.VMEM),
            pl.BlockSpec(memory_space=pltpu.VMEM),
            pl.BlockSpec(memory_space=pltpu.VMEM),
            pl.BlockSpec(memory_space=pltpu.VMEM),
            pl.BlockSpec(memory_space=pltpu.SMEM),
            pl.BlockSpec(memory_space=pltpu.SMEM),
        ],
        out_specs=pl.BlockSpec(memory_space=pltpu.HBM),
        out_shape=jax.ShapeDtypeStruct((rows, h_dim), jnp.float32),
        scratch_shapes=[
            pltpu.VMEM((TILE, px.shape[1]), jnp.float32),
            pltpu.VMEM((TILE, px.shape[1]), jnp.float32),
            pltpu.VMEM((TILE, h_dim), jnp.float32),
            pltpu.VMEM((TILE, h_dim), jnp.float32),
            pltpu.VMEM((TILE, h_dim), jnp.float32),
            pltpu.VMEM((TILE, h_dim), jnp.float32),
            pltpu.SemaphoreType.DMA,
            pltpu.SemaphoreType.DMA,
            pltpu.SemaphoreType.DMA,
            pltpu.SemaphoreType.DMA,
            pltpu.SemaphoreType.DMA,
            pltpu.SemaphoreType.DMA,
        ],
    )(px, wih_t, whh_t, bih2, bhh2, h0, m0col, meta, lanes)


def kernel(x, hidden_states, masks, W_ih, W_hh, b_ih, b_hh):
    n = hidden_states.shape[1]
    h_dim = hidden_states.shape[2]
    t = x.shape[0] // n
    r_tot = t * n

    ms = masks.reshape(t, n)
    masks_nt = ms.T.astype(jnp.int32).reshape(r_tot)
    perm_nt, meta, h0lanes = _build_index(masks_nt, n)
    perm_r = perm_nt.reshape(n, t).T.reshape(r_tot)
    lanes16 = h0lanes.reshape(n, L)[:, 1]

    r_pad = r_tot + 8 * t
    packed_x = _scatter_rows(x, perm_r, r_pad + TILE)

    wih_t = W_ih.T
    whh_t = W_hh.T
    bih2 = b_ih.reshape(1, 3 * h_dim)
    bhh2 = b_hh.reshape(1, 3 * h_dim)
    h0 = hidden_states[0]
    m0col = jnp.broadcast_to(
        ms[0].astype(jnp.float32).reshape(n, 1), (n, 128))

    packed_out = _packed_scan(packed_x, wih_t, whh_t, bih2, bhh2, h0,
                              m0col, meta, lanes16)

    out = _gather_rows(packed_out, perm_r, r_tot)
    h_final = out.reshape(t, n, h_dim)[-1][None]
    return out, h_final


# speedup vs baseline: 1.0163x; 1.0087x over previous
import functools

import jax
import jax.numpy as jnp
from jax import lax
from jax.experimental import pallas as pl
from jax.experimental.pallas import tpu as pltpu
from jax.experimental.pallas import tpu_sc as plsc

L = 16
TILE = 256
GCH = 128


def _index_kernel_body(t_len, n_env, masks_hbm, perm_hbm, meta_hbm,
                       h0lane_hbm, m_v, rel_v, eid_v, pos_v, len_v, lane_v,
                       hist_v, b_v, off_v, occ_v, meta_v, row_v, hist_sh):
    cid = lax.axis_index("c")
    sid = lax.axis_index("s")
    nchunks = t_len // L
    tbl_ch = (t_len + 2 * L) // L

    @pl.when(cid == 0)
    def _worker():
        env = sid
        pltpu.sync_copy(masks_hbm.at[pl.ds(env * t_len, t_len)], m_v)

        zeros = jnp.zeros((L,), jnp.int32)
        def zero_body(i, _):
            hist_v[pl.ds(i * L, L)] = zeros
            occ_v[pl.ds(i * L, L)] = zeros
            return 0
        lax.fori_loop(0, tbl_ch, zero_body, 0)

        iota = lax.iota(jnp.int32, L)

        def p1(i, carry):
            eid_c, ls_c = carry
            tv = iota + i * L
            m = m_v[pl.ds(i * L, L)]
            st = jnp.where((tv == 0) | (m == 0), 1, 0).astype(jnp.int32)
            eidv = plsc.cumsum(st) + eid_c
            lsv = jnp.maximum(plsc.cummax(jnp.where(st == 1, tv, -1)), ls_c)
            rel_v[pl.ds(i * L, L)] = tv - lsv
            eid_v[pl.ds(i * L, L)] = eidv
            plsc.store_scatter(pos_v, [eidv], tv, mask=st == 1)
            return (jnp.max(eidv), jnp.max(lsv))
        e_cnt, _ = lax.fori_loop(0, nchunks, p1, (jnp.int32(0), jnp.int32(-1)))

        def p2(i, _):
            ev = iota + 1 + i * L
            valid = ev <= e_cnt
            p_here = plsc.load_gather(pos_v, [jnp.where(valid, ev, 0)])
            nxt = plsc.load_gather(pos_v, [jnp.where(ev < e_cnt, ev + 1, 0)])
            lenv = jnp.where(ev == e_cnt, t_len - p_here, nxt - p_here)
            lenv = jnp.where(valid, lenv, 0)
            plsc.store_scatter(len_v, [ev], lenv, mask=valid)
            plsc.addupdate_scatter(hist_v, [lenv],
                                   jnp.ones((L,), jnp.int32), mask=valid)
            return 0
        lax.fori_loop(0, nchunks, p2, 0)

        pltpu.sync_copy(hist_v, hist_sh.at[env])
        plsc.subcore_barrier()

        def add_env(e, _):
            pltpu.sync_copy(hist_sh.at[e], meta_v)
            def addc(i, _):
                hv = meta_v[pl.ds(i * L, L)]
                hist_v[pl.ds(i * L, L)] = hist_v[pl.ds(i * L, L)] + hv
                occ_v[pl.ds(i * L, L)] = occ_v[pl.ds(i * L, L)] + jnp.where(
                    e < env, hv, 0)
                return 0
            lax.fori_loop(0, tbl_ch, addc, 0)
            return 0
        def zero_hist(i, _):
            hist_v[pl.ds(i * L, L)] = zeros
            return 0
        lax.fori_loop(0, tbl_ch, zero_hist, 0)
        lax.fori_loop(0, n_env, add_env, 0)

        def p3(i, carry):
            tot, mx_c = carry
            hv = hist_v[pl.ds(i * L, L)]
            pref = plsc.cumsum(hv) + tot
            b_v[pl.ds(i * L, L)] = pref
            sv = iota + i * L
            mx = jnp.max(jnp.where(hv > 0, sv, 0))
            return (jnp.max(pref), jnp.maximum(mx_c, mx))
        carry = lax.fori_loop(0, tbl_ch, p3, (jnp.int32(0), jnp.int32(0)))
        e_tot, s_max = carry
        def p3b(i, _):
            b_v[pl.ds(i * L, L)] = e_tot - b_v[pl.ds(i * L, L)]
            return 0
        lax.fori_loop(0, tbl_ch, p3b, 0)

        def p3c(i, off_c):
            bv = b_v[pl.ds(i * L, L)]
            rb = (bv + 7) // 8 * 8
            cs = plsc.cumsum(rb)
            off_v[pl.ds(i * L, L)] = cs - rb + off_c
            return off_c + cs[L - 1]
        lax.fori_loop(0, tbl_ch, p3c, jnp.int32(0))

        lane0 = iota == 0
        def p4(e, _):
            e_vec = jnp.zeros((L,), jnp.int32) + e
            ln_vec = plsc.load_gather(len_v, [e_vec])
            b_l = plsc.load_gather(b_v, [ln_vec])
            occ_l = plsc.load_gather(occ_v, [ln_vec])
            plsc.store_scatter(lane_v, [e_vec], b_l + occ_l, mask=lane0)
            plsc.store_scatter(occ_v, [ln_vec], occ_l + 1, mask=lane0)
            return 0
        lax.fori_loop(1, e_cnt + 1, p4, 0)

        def p5(i, _):
            relv = rel_v[pl.ds(i * L, L)]
            eidv = eid_v[pl.ds(i * L, L)]
            lanes = plsc.load_gather(lane_v, [eidv])
            offs = plsc.load_gather(off_v, [relv])
            row_v[pl.ds(i * L, L)] = offs + lanes
            return 0
        lax.fori_loop(0, nchunks, p5, 0)
        pltpu.sync_copy(row_v, perm_hbm.at[pl.ds(env * t_len, t_len)])

        pltpu.sync_copy(lane_v.at[pl.ds(0, L)], h0lane_hbm.at[pl.ds(env * L, L)])

        @pl.when(env == 0)
        def _meta():
            meta_v[pl.ds(0, L)] = jnp.where(iota == 0, s_max, 0)
            def mcopy(i, _):
                ov = off_v[pl.ds(i * L, L)]
                sv = iota + i * L
                plsc.store_scatter(meta_v, [sv + 1], ov,
                                   mask=sv <= t_len)
                return 0
            lax.fori_loop(0, tbl_ch, mcopy, 0)
            pltpu.sync_copy(meta_v, meta_hbm)


def _build_index(masks_nt, n_env):
    t_len = masks_nt.shape[0] // n_env
    r_tot = t_len * n_env
    tbl = ((t_len + 2 * L) // L) * L
    mesh = plsc.VectorSubcoreMesh(core_axis_name="c", subcore_axis_name="s")

    body = functools.partial(_index_kernel_body, t_len, n_env)
    kern = functools.partial(
        pl.kernel, mesh=mesh,
        compiler_params=pltpu.CompilerParams(needs_layout_passes=False),
        out_type=(
            jax.ShapeDtypeStruct((r_tot,), jnp.int32),
            jax.ShapeDtypeStruct((tbl,), jnp.int32),
            jax.ShapeDtypeStruct((n_env * L,), jnp.int32),
        ),
        scratch_types=[
            pltpu.VMEM((t_len,), jnp.int32),
            pltpu.VMEM((t_len,), jnp.int32),
            pltpu.VMEM((t_len,), jnp.int32),
            pltpu.VMEM((tbl,), jnp.int32),
            pltpu.VMEM((tbl,), jnp.int32),
            pltpu.VMEM((tbl,), jnp.int32),
            pltpu.VMEM((tbl,), jnp.int32),
            pltpu.VMEM((tbl,), jnp.int32),
            pltpu.VMEM((tbl,), jnp.int32),
            pltpu.VMEM((tbl,), jnp.int32),
            pltpu.VMEM((tbl,), jnp.int32),
            pltpu.VMEM((t_len,), jnp.int32),
            pltpu.VMEM_SHARED((n_env, tbl), jnp.int32),
        ],
    )(body)
    return kern(masks_nt)


def _scatter_rows(rows_in, idx, out_rows):
    n_idx = idx.shape[0]
    d = rows_in.shape[1]
    mesh = plsc.VectorSubcoreMesh(core_axis_name="c", subcore_axis_name="s")
    info = plsc.get_sparse_core_info()
    nw = info.num_cores * info.num_subcores
    per_w = n_idx // nw
    n_ch = per_w // GCH

    def body(rows_hbm, idx_hbm, out_hbm, idx2_v, rows_v, sem):
        wid = lax.axis_index("s") * info.num_cores + lax.axis_index("c")
        base = wid * per_w
        def ld_idx(j, _):
            pltpu.sync_copy(idx_hbm.at[pl.ds(base + j * GCH, GCH)],
                            idx2_v.at[j])
            return 0
        lax.fori_loop(0, n_ch, ld_idx, 0)
        def chunk(j, _):
            pltpu.sync_copy(rows_hbm.at[pl.ds(base + j * GCH, GCH)], rows_v)
            pltpu.async_copy(rows_v, out_hbm.at[idx2_v.at[j]], sem).wait()
            return 0
        lax.fori_loop(0, n_ch, chunk, 0)

    kern = functools.partial(
        pl.kernel, mesh=mesh,
        compiler_params=pltpu.CompilerParams(needs_layout_passes=False),
        out_type=jax.ShapeDtypeStruct((out_rows, d), jnp.float32),
        scratch_types=[
            pltpu.VMEM((per_w // GCH, GCH), jnp.int32),
            pltpu.VMEM((GCH, d), jnp.float32),
            pltpu.SemaphoreType.DMA,
        ],
    )(body)
    return kern(rows_in, idx)


def _gather_rows(table, idx, out_rows):
    n_idx = idx.shape[0]
    d = table.shape[1]
    mesh = plsc.VectorSubcoreMesh(core_axis_name="c", subcore_axis_name="s")
    info = plsc.get_sparse_core_info()
    nw = info.num_cores * info.num_subcores
    per_w = n_idx // nw

    def body(table_hbm, idx_hbm, out_hbm, idx_v, rows_v, sem):
        wid = lax.axis_index("s") * info.num_cores + lax.axis_index("c")
        base = wid * per_w
        def chunk(i, _):
            b = base + i * GCH
            pltpu.sync_copy(idx_hbm.at[pl.ds(b, GCH)], idx_v)
            pltpu.async_copy(table_hbm.at[idx_v], rows_v, sem).wait()
            pltpu.sync_copy(rows_v, out_hbm.at[pl.ds(b, GCH)])
            return 0
        lax.fori_loop(0, per_w // GCH, chunk, 0)

    kern = functools.partial(
        pl.kernel, mesh=mesh,
        compiler_params=pltpu.CompilerParams(needs_layout_passes=False),
        out_type=jax.ShapeDtypeStruct((out_rows, d), jnp.float32),
        scratch_types=[
            pltpu.VMEM((GCH,), jnp.int32),
            pltpu.VMEM((GCH, d), jnp.float32),
            pltpu.SemaphoreType.DMA,
        ],
    )(body)
    return kern(table, idx)


def _scan_body(n_env, h_dim, px_hbm, wih_ref, whh_ref, bih_ref,
               bhh_ref, h0_ref, m0_ref, meta_ref, lanes_ref, pout_hbm,
               xa_ref, xb_ref, ha_ref, hb_ref, oa_ref, ob_ref,
               sx0, sx1, sh0, sh1, so0, so1):
    s_max = meta_ref[0]
    wih = wih_ref[...].astype(jnp.bfloat16)
    whh = whh_ref[...].astype(jnp.bfloat16)
    bih = bih_ref[...]
    bhh = bhh_ref[...]

    def xcp(base, buf, sem):
        return pltpu.make_async_copy(
            px_hbm.at[pl.ds(pl.multiple_of(base, 8), TILE)], buf, sem)

    def hcp(base, buf, sem):
        return pltpu.make_async_copy(
            pout_hbm.at[pl.ds(pl.multiple_of(base, 8), TILE)], buf, sem)

    def ocp(buf, base, sem):
        return pltpu.make_async_copy(
            buf, pout_hbm.at[pl.ds(pl.multiple_of(base, 8), TILE)], sem)

    def step(s, _):
        off_s = meta_ref[1 + s]
        b_s = meta_ref[2 + s] - off_s
        off_p = meta_ref[s]
        nb = (b_s + TILE - 1) // TILE

        xcp(off_s, xa_ref, sx0).start()
        @pl.when(s > 0)
        def _():
            hcp(off_p, ha_ref, sh0).start()

        def tile(tb, _):
            base = pl.multiple_of(off_s + tb * TILE, 8)
            even = tb % 2 == 0

            @pl.when(tb + 1 < nb)
            def _prefetch():
                nx = off_s + (tb + 1) * TILE
                nh = off_p + (tb + 1) * TILE
                @pl.when(even)
                def _():
                    xcp(nx, xb_ref, sx1).start()
                    @pl.when(s > 0)
                    def _():
                        hcp(nh, hb_ref, sh1).start()
                @pl.when(jnp.logical_not(even))
                def _():
                    xcp(nx, xa_ref, sx0).start()
                    @pl.when(s > 0)
                    def _():
                        hcp(nh, ha_ref, sh0).start()

            def gru(xv, hp):
                gi = jnp.dot(xv.astype(jnp.bfloat16), wih,
                             preferred_element_type=jnp.float32) + bih
                gh = jnp.dot(hp.astype(jnp.bfloat16), whh,
                             preferred_element_type=jnp.float32) + bhh
                r = jax.nn.sigmoid(gi[:, :h_dim] + gh[:, :h_dim])
                z = jax.nn.sigmoid(
                    gi[:, h_dim:2 * h_dim] + gh[:, h_dim:2 * h_dim])
                ng = jnp.tanh(gi[:, 2 * h_dim:] + r * gh[:, 2 * h_dim:])
                return (1.0 - z) * ng + z * hp

            def proc(xr, hr, orr, sx, sh, so):
                @pl.when(s > 0)
                def _():
                    hcp(base, hr, sh).wait()
                @pl.when(s == 0)
                def _init():
                    hr[...] = jnp.zeros((TILE, h_dim), jnp.float32)
                    for k in range(n_env):
                        lane = lanes_ref[k]
                        @pl.when((lane >= tb * TILE) &
                                 (lane < tb * TILE + TILE))
                        def _():
                            hr[pl.ds(lane - tb * TILE, 1), :] = (
                                h0_ref[k:k + 1, :] * m0_ref[k:k + 1, 0:1])
                xcp(base, xr, sx).wait()
                @pl.when(tb >= 2)
                def _():
                    ocp(orr, base, so).wait()
                @pl.when((s > 0) & (b_s <= 64))
                def _():
                    orr[0:64, :] = gru(xr[0:64, :], hr[0:64, :])
                @pl.when((s == 0) | (b_s > 64))
                def _():
                    orr[...] = gru(xr[...], hr[...])
                ocp(orr, base, so).start()

            @pl.when(even)
            def _():
                proc(xa_ref, ha_ref, oa_ref, sx0, sh0, so0)
            @pl.when(jnp.logical_not(even))
            def _():
                proc(xb_ref, hb_ref, ob_ref, sx1, sh1, so1)
            return 0

        lax.fori_loop(0, nb, tile, 0)

        @pl.when(nb % 2 == 1)
        def _():
            ocp(oa_ref, off_s, so0).wait()
            @pl.when(nb >= 2)
            def _():
                ocp(ob_ref, off_s, so1).wait()
        @pl.when(nb % 2 == 0)
        def _():
            ocp(ob_ref, off_s, so1).wait()
            @pl.when(nb >= 2)
            def _():
                ocp(oa_ref, off_s, so0).wait()
        return 0

    lax.fori_loop(0, s_max, step, 0)


def _packed_scan(px, wih_t, whh_t, bih2, bhh2, h0, m0col, meta, lanes):
    rows = px.shape[0]
    h_dim = whh_t.shape[0]
    n_env = h0.shape[0]
    body = functools.partial(_scan_body, n_env, h_dim)
    return pl.pallas_call(
        body,
        in_specs=[
            pl.BlockSpec(memory_space=pltpu.HBM),
            pl.BlockSpec(memory_space=pltpu.VMEM),
            pl.BlockSpec(memory_space=pltpu.VMEM),
            pl.BlockSpec(memory_space=pltpu.VMEM),
            pl.BlockSpec(memory_space=pltpu.VMEM),
            pl.BlockSpec(memory_space=pltpu.VMEM),
            pl.BlockSpec(memory_space=pltpu.VMEM),
            pl.BlockSpec(memory_space=pltpu.SMEM),
            pl.BlockSpec(memory_space=pltpu.SMEM),
        ],
        out_specs=pl.BlockSpec(memory_space=pltpu.HBM),
        out_shape=jax.ShapeDtypeStruct((rows, h_dim), jnp.float32),
        scratch_shapes=[
            pltpu.VMEM((TILE, px.shape[1]), jnp.float32),
            pltpu.VMEM((TILE, px.shape[1]), jnp.float32),
            pltpu.VMEM((TILE, h_dim), jnp.float32),
            pltpu.VMEM((TILE, h_dim), jnp.float32),
            pltpu.VMEM((TILE, h_dim), jnp.float32),
            pltpu.VMEM((TILE, h_dim), jnp.float32),
            pltpu.SemaphoreType.DMA,
            pltpu.SemaphoreType.DMA,
            pltpu.SemaphoreType.DMA,
            pltpu.SemaphoreType.DMA,
            pltpu.SemaphoreType.DMA,
            pltpu.SemaphoreType.DMA,
        ],
    )(px, wih_t, whh_t, bih2, bhh2, h0, m0col, meta, lanes)


def kernel(x, hidden_states, masks, W_ih, W_hh, b_ih, b_hh):
    n = hidden_states.shape[1]
    h_dim = hidden_states.shape[2]
    t = x.shape[0] // n
    r_tot = t * n

    ms = masks.reshape(t, n)
    masks_nt = ms.T.astype(jnp.int32).reshape(r_tot)
    perm_nt, meta, h0lanes = _build_index(masks_nt, n)
    perm_r = perm_nt.reshape(n, t).T.reshape(r_tot)
    lanes16 = h0lanes.reshape(n, L)[:, 1]

    r_pad = r_tot + 8 * t
    packed_x = _scatter_rows(x, perm_r, r_pad + TILE)

    wih_t = W_ih.T
    whh_t = W_hh.T
    bih2 = b_ih.reshape(1, 3 * h_dim)
    bhh2 = b_hh.reshape(1, 3 * h_dim)
    h0 = hidden_states[0]
    m0col = jnp.broadcast_to(
        ms[0].astype(jnp.float32).reshape(n, 1), (n, 128))

    packed_out = _packed_scan(packed_x, wih_t, whh_t, bih2, bhh2, h0,
                              m0col, meta, lanes16)

    out = _gather_rows(packed_out, perm_r, r_tot)
    h_final = out.reshape(t, n, h_dim)[-1][None]
    return out, h_final


# speedup vs baseline: 1.0177x; 1.0014x over previous
import functools

import jax
import jax.numpy as jnp
from jax import lax
from jax.experimental import pallas as pl
from jax.experimental.pallas import tpu as pltpu
from jax.experimental.pallas import tpu_sc as plsc

L = 16
TILE = 256
GCH = 128


def _index_kernel_body(t_len, n_env, masks_hbm, perm_hbm, meta_hbm,
                       h0lane_hbm, m_v, rel_v, eid_v, pos_v, len_v, lane_v,
                       hist_v, b_v, off_v, occ_v, meta_v, row_v, hist_sh):
    cid = lax.axis_index("c")
    sid = lax.axis_index("s")
    nchunks = t_len // L
    tbl_ch = (t_len + 2 * L) // L

    @pl.when(cid == 0)
    def _worker():
        env = sid
        pltpu.sync_copy(masks_hbm.at[pl.ds(env * t_len, t_len)], m_v)

        zeros = jnp.zeros((L,), jnp.int32)
        def zero_body(i, _):
            hist_v[pl.ds(i * L, L)] = zeros
            occ_v[pl.ds(i * L, L)] = zeros
            return 0
        lax.fori_loop(0, tbl_ch, zero_body, 0)

        iota = lax.iota(jnp.int32, L)

        def p1(i, carry):
            eid_c, ls_c = carry
            tv = iota + i * L
            m = m_v[pl.ds(i * L, L)]
            st = jnp.where((tv == 0) | (m == 0), 1, 0).astype(jnp.int32)
            eidv = plsc.cumsum(st) + eid_c
            lsv = jnp.maximum(plsc.cummax(jnp.where(st == 1, tv, -1)), ls_c)
            rel_v[pl.ds(i * L, L)] = tv - lsv
            eid_v[pl.ds(i * L, L)] = eidv
            plsc.store_scatter(pos_v, [eidv], tv, mask=st == 1)
            return (jnp.max(eidv), jnp.max(lsv))
        e_cnt, _ = lax.fori_loop(0, nchunks, p1, (jnp.int32(0), jnp.int32(-1)))

        def p2(i, _):
            ev = iota + 1 + i * L
            valid = ev <= e_cnt
            p_here = plsc.load_gather(pos_v, [jnp.where(valid, ev, 0)])
            nxt = plsc.load_gather(pos_v, [jnp.where(ev < e_cnt, ev + 1, 0)])
            lenv = jnp.where(ev == e_cnt, t_len - p_here, nxt - p_here)
            lenv = jnp.where(valid, lenv, 0)
            plsc.store_scatter(len_v, [ev], lenv, mask=valid)
            plsc.addupdate_scatter(hist_v, [lenv],
                                   jnp.ones((L,), jnp.int32), mask=valid)
            return 0
        lax.fori_loop(0, nchunks, p2, 0)

        pltpu.sync_copy(hist_v, hist_sh.at[env])
        plsc.subcore_barrier()

        def add_env(e, _):
            pltpu.sync_copy(hist_sh.at[e], meta_v)
            def addc(i, _):
                hv = meta_v[pl.ds(i * L, L)]
                hist_v[pl.ds(i * L, L)] = hist_v[pl.ds(i * L, L)] + hv
                occ_v[pl.ds(i * L, L)] = occ_v[pl.ds(i * L, L)] + jnp.where(
                    e < env, hv, 0)
                return 0
            lax.fori_loop(0, tbl_ch, addc, 0)
            return 0
        def zero_hist(i, _):
            hist_v[pl.ds(i * L, L)] = zeros
            return 0
        lax.fori_loop(0, tbl_ch, zero_hist, 0)
        lax.fori_loop(0, n_env, add_env, 0)

        def p3(i, carry):
            tot, mx_c = carry
            hv = hist_v[pl.ds(i * L, L)]
            pref = plsc.cumsum(hv) + tot
            b_v[pl.ds(i * L, L)] = pref
            sv = iota + i * L
            mx = jnp.max(jnp.where(hv > 0, sv, 0))
            return (jnp.max(pref), jnp.maximum(mx_c, mx))
        carry = lax.fori_loop(0, tbl_ch, p3, (jnp.int32(0), jnp.int32(0)))
        e_tot, s_max = carry
        def p3b(i, _):
            b_v[pl.ds(i * L, L)] = e_tot - b_v[pl.ds(i * L, L)]
            return 0
        lax.fori_loop(0, tbl_ch, p3b, 0)

        def p3c(i, off_c):
            bv = b_v[pl.ds(i * L, L)]
            rb = (bv + 7) // 8 * 8
            cs = plsc.cumsum(rb)
            off_v[pl.ds(i * L, L)] = cs - rb + off_c
            return off_c + cs[L - 1]
        lax.fori_loop(0, tbl_ch, p3c, jnp.int32(0))

        lane0 = iota == 0
        def p4(e, _):
            e_vec = jnp.zeros((L,), jnp.int32) + e
            ln_vec = plsc.load_gather(len_v, [e_vec])
            b_l = plsc.load_gather(b_v, [ln_vec])
            occ_l = plsc.load_gather(occ_v, [ln_vec])
            plsc.store_scatter(lane_v, [e_vec], b_l + occ_l, mask=lane0)
            plsc.store_scatter(occ_v, [ln_vec], occ_l + 1, mask=lane0)
            return 0
        lax.fori_loop(1, e_cnt + 1, p4, 0)

        def p5(i, _):
            relv = rel_v[pl.ds(i * L, L)]
            eidv = eid_v[pl.ds(i * L, L)]
            lanes = plsc.load_gather(lane_v, [eidv])
            offs = plsc.load_gather(off_v, [relv])
            row_v[pl.ds(i * L, L)] = offs + lanes
            return 0
        lax.fori_loop(0, nchunks, p5, 0)
        pltpu.sync_copy(row_v, perm_hbm.at[pl.ds(env * t_len, t_len)])

        pltpu.sync_copy(lane_v.at[pl.ds(0, L)], h0lane_hbm.at[pl.ds(env * L, L)])

        @pl.when(env == 0)
        def _meta():
            meta_v[pl.ds(0, L)] = jnp.where(iota == 0, s_max, 0)
            def mcopy(i, _):
                ov = off_v[pl.ds(i * L, L)]
                sv = iota + i * L
                plsc.store_scatter(meta_v, [sv + 1], ov,
                                   mask=sv <= t_len)
                return 0
            lax.fori_loop(0, tbl_ch, mcopy, 0)
            pltpu.sync_copy(meta_v, meta_hbm)


def _build_index(masks_nt, n_env):
    t_len = masks_nt.shape[0] // n_env
    r_tot = t_len * n_env
    tbl = ((t_len + 2 * L) // L) * L
    mesh = plsc.VectorSubcoreMesh(core_axis_name="c", subcore_axis_name="s")

    body = functools.partial(_index_kernel_body, t_len, n_env)
    kern = functools.partial(
        pl.kernel, mesh=mesh,
        compiler_params=pltpu.CompilerParams(needs_layout_passes=False),
        out_type=(
            jax.ShapeDtypeStruct((r_tot,), jnp.int32),
            jax.ShapeDtypeStruct((tbl,), jnp.int32),
            jax.ShapeDtypeStruct((n_env * L,), jnp.int32),
        ),
        scratch_types=[
            pltpu.VMEM((t_len,), jnp.int32),
            pltpu.VMEM((t_len,), jnp.int32),
            pltpu.VMEM((t_len,), jnp.int32),
            pltpu.VMEM((tbl,), jnp.int32),
            pltpu.VMEM((tbl,), jnp.int32),
            pltpu.VMEM((tbl,), jnp.int32),
            pltpu.VMEM((tbl,), jnp.int32),
            pltpu.VMEM((tbl,), jnp.int32),
            pltpu.VMEM((tbl,), jnp.int32),
            pltpu.VMEM((tbl,), jnp.int32),
            pltpu.VMEM((tbl,), jnp.int32),
            pltpu.VMEM((t_len,), jnp.int32),
            pltpu.VMEM_SHARED((n_env, tbl), jnp.int32),
        ],
    )(body)
    return kern(masks_nt)


def _scatter_rows(rows_in, idx, out_rows):
    n_idx = idx.shape[0]
    d = rows_in.shape[1]
    mesh = plsc.VectorSubcoreMesh(core_axis_name="c", subcore_axis_name="s")
    info = plsc.get_sparse_core_info()
    nw = info.num_cores * info.num_subcores
    per_w = n_idx // nw
    n_ch = per_w // GCH

    def body(rows_hbm, idx_hbm, out_hbm, idx2_v, rows_v, sem):
        wid = lax.axis_index("s") * info.num_cores + lax.axis_index("c")
        base = wid * per_w
        def ld_idx(j, _):
            pltpu.sync_copy(idx_hbm.at[pl.ds(base + j * GCH, GCH)],
                            idx2_v.at[j])
            return 0
        lax.fori_loop(0, n_ch, ld_idx, 0)
        def chunk(j, _):
            pltpu.sync_copy(rows_hbm.at[pl.ds(base + j * GCH, GCH)], rows_v)
            pltpu.async_copy(rows_v, out_hbm.at[idx2_v.at[j]], sem).wait()
            return 0
        lax.fori_loop(0, n_ch, chunk, 0)

    kern = functools.partial(
        pl.kernel, mesh=mesh,
        compiler_params=pltpu.CompilerParams(needs_layout_passes=False),
        out_type=jax.ShapeDtypeStruct((out_rows, d), jnp.float32),
        scratch_types=[
            pltpu.VMEM((per_w // GCH, GCH), jnp.int32),
            pltpu.VMEM((GCH, d), jnp.float32),
            pltpu.SemaphoreType.DMA,
        ],
    )(body)
    return kern(rows_in, idx)


def _gather_rows(table, idx, out_rows):
    n_idx = idx.shape[0]
    d = table.shape[1]
    mesh = plsc.VectorSubcoreMesh(core_axis_name="c", subcore_axis_name="s")
    info = plsc.get_sparse_core_info()
    nw = info.num_cores * info.num_subcores
    per_w = n_idx // nw

    def body(table_hbm, idx_hbm, out_hbm, idx_v, rows_v, sem):
        wid = lax.axis_index("s") * info.num_cores + lax.axis_index("c")
        base = wid * per_w
        def chunk(i, _):
            b = base + i * GCH
            pltpu.sync_copy(idx_hbm.at[pl.ds(b, GCH)], idx_v)
            pltpu.async_copy(table_hbm.at[idx_v], rows_v, sem).wait()
            pltpu.sync_copy(rows_v, out_hbm.at[pl.ds(b, GCH)])
            return 0
        lax.fori_loop(0, per_w // GCH, chunk, 0)

    kern = functools.partial(
        pl.kernel, mesh=mesh,
        compiler_params=pltpu.CompilerParams(needs_layout_passes=False),
        out_type=jax.ShapeDtypeStruct((out_rows, d), jnp.float32),
        scratch_types=[
            pltpu.VMEM((GCH,), jnp.int32),
            pltpu.VMEM((GCH, d), jnp.float32),
            pltpu.SemaphoreType.DMA,
        ],
    )(body)
    return kern(table, idx)


def _scan_body(n_env, h_dim, px_hbm, wih_ref, whh_ref, bih_ref,
               bhh_ref, h0_ref, m0_ref, meta_ref, lanes_ref, pout_hbm,
               xa_ref, xb_ref, ha_ref, hb_ref, oa_ref, ob_ref,
               sx0, sx1, sh0, sh1, so0, so1):
    s_max = meta_ref[0]
    wih = wih_ref[...].astype(jnp.bfloat16)
    whh = whh_ref[...].astype(jnp.bfloat16)
    bih = bih_ref[...]
    bhh = bhh_ref[...]

    def xcp(base, buf, sem):
        return pltpu.make_async_copy(
            px_hbm.at[pl.ds(pl.multiple_of(base, 8), TILE)], buf, sem)

    def hcp(base, buf, sem):
        return pltpu.make_async_copy(
            pout_hbm.at[pl.ds(pl.multiple_of(base, 8), TILE)], buf, sem)

    def ocp(buf, base, sem):
        return pltpu.make_async_copy(
            buf, pout_hbm.at[pl.ds(pl.multiple_of(base, 8), TILE)], sem)

    def step(s, _):
        off_s = meta_ref[1 + s]
        b_s = meta_ref[2 + s] - off_s
        off_p = meta_ref[s]
        nb = (b_s + TILE - 1) // TILE

        xcp(off_s, xa_ref, sx0).start()
        @pl.when(s > 0)
        def _():
            hcp(off_p, ha_ref, sh0).start()

        def tile(tb, _):
            base = pl.multiple_of(off_s + tb * TILE, 8)
            even = tb % 2 == 0

            @pl.when(tb + 1 < nb)
            def _prefetch():
                nx = off_s + (tb + 1) * TILE
                nh = off_p + (tb + 1) * TILE
                @pl.when(even)
                def _():
                    xcp(nx, xb_ref, sx1).start()
                    @pl.when(s > 0)
                    def _():
                        hcp(nh, hb_ref, sh1).start()
                @pl.when(jnp.logical_not(even))
                def _():
                    xcp(nx, xa_ref, sx0).start()
                    @pl.when(s > 0)
                    def _():
                        hcp(nh, ha_ref, sh0).start()

            def gru(xv, hp):
                gi = jnp.dot(xv.astype(jnp.bfloat16), wih,
                             preferred_element_type=jnp.float32) + bih
                gh = jnp.dot(hp.astype(jnp.bfloat16), whh,
                             preferred_element_type=jnp.float32) + bhh
                r = jax.nn.sigmoid(gi[:, :h_dim] + gh[:, :h_dim])
                z = jax.nn.sigmoid(
                    gi[:, h_dim:2 * h_dim] + gh[:, h_dim:2 * h_dim])
                ng = jnp.tanh(gi[:, 2 * h_dim:] + r * gh[:, 2 * h_dim:])
                return (1.0 - z) * ng + z * hp

            def proc(xr, hr, orr, sx, sh, so):
                @pl.when(s > 0)
                def _():
                    hcp(base, hr, sh).wait()
                @pl.when(s == 0)
                def _init():
                    hr[...] = jnp.zeros((TILE, h_dim), jnp.float32)
                    for k in range(n_env):
                        lane = lanes_ref[k]
                        @pl.when((lane >= tb * TILE) &
                                 (lane < tb * TILE + TILE))
                        def _():
                            hr[pl.ds(lane - tb * TILE, 1), :] = (
                                h0_ref[k:k + 1, :] * m0_ref[k:k + 1, 0:1])
                xcp(base, xr, sx).wait()
                @pl.when(tb >= 2)
                def _():
                    ocp(orr, base, so).wait()
                @pl.when((s > 0) & (b_s <= 128))
                def _():
                    orr[0:128, :] = gru(xr[0:128, :], hr[0:128, :])
                @pl.when((s == 0) | (b_s > 128))
                def _():
                    orr[...] = gru(xr[...], hr[...])
                ocp(orr, base, so).start()

            @pl.when(even)
            def _():
                proc(xa_ref, ha_ref, oa_ref, sx0, sh0, so0)
            @pl.when(jnp.logical_not(even))
            def _():
                proc(xb_ref, hb_ref, ob_ref, sx1, sh1, so1)
            return 0

        lax.fori_loop(0, nb, tile, 0)

        @pl.when(nb % 2 == 1)
        def _():
            ocp(oa_ref, off_s, so0).wait()
            @pl.when(nb >= 2)
            def _():
                ocp(ob_ref, off_s, so1).wait()
        @pl.when(nb % 2 == 0)
        def _():
            ocp(ob_ref, off_s, so1).wait()
            @pl.when(nb >= 2)
            def _():
                ocp(oa_ref, off_s, so0).wait()
        return 0

    lax.fori_loop(0, s_max, step, 0)


def _packed_scan(px, wih_t, whh_t, bih2, bhh2, h0, m0col, meta, lanes):
    rows = px.shape[0]
    h_dim = whh_t.shape[0]
    n_env = h0.shape[0]
    body = functools.partial(_scan_body, n_env, h_dim)
    return pl.pallas_call(
        body,
        in_specs=[
            pl.BlockSpec(memory_space=pltpu.HBM),
            pl.BlockSpec(memory_space=pltpu.VMEM),
            pl.BlockSpec(memory_space=pltpu.VMEM),
            pl.BlockSpec(memory_space=pltpu.VMEM),
            pl.BlockSpec(memory_space=pltpu.VMEM),
            pl.BlockSpec(memory_space=pltpu.VMEM),
            pl.BlockSpec(memory_space=pltpu.VMEM),
            pl.BlockSpec(memory_space=pltpu.SMEM),
            pl.BlockSpec(memory_space=pltpu.SMEM),
        ],
        out_specs=pl.BlockSpec(memory_space=pltpu.HBM),
        out_shape=jax.ShapeDtypeStruct((rows, h_dim), jnp.float32),
        scratch_shapes=[
            pltpu.VMEM((TILE, px.shape[1]), jnp.float32),
            pltpu.VMEM((TILE, px.shape[1]), jnp.float32),
            pltpu.VMEM((TILE, h_dim), jnp.float32),
            pltpu.VMEM((TILE, h_dim), jnp.float32),
            pltpu.VMEM((TILE, h_dim), jnp.float32),
            pltpu.VMEM((TILE, h_dim), jnp.float32),
            pltpu.SemaphoreType.DMA,
            pltpu.SemaphoreType.DMA,
            pltpu.SemaphoreType.DMA,
            pltpu.SemaphoreType.DMA,
            pltpu.SemaphoreType.DMA,
            pltpu.SemaphoreType.DMA,
        ],
    )(px, wih_t, whh_t, bih2, bhh2, h0, m0col, meta, lanes)


def kernel(x, hidden_states, masks, W_ih, W_hh, b_ih, b_hh):
    n = hidden_states.shape[1]
    h_dim = hidden_states.shape[2]
    t = x.shape[0] // n
    r_tot = t * n

    ms = masks.reshape(t, n)
    masks_nt = ms.T.astype(jnp.int32).reshape(r_tot)
    perm_nt, meta, h0lanes = _build_index(masks_nt, n)
    perm_r = perm_nt.reshape(n, t).T.reshape(r_tot)
    lanes16 = h0lanes.reshape(n, L)[:, 1]

    r_pad = r_tot + 8 * t
    packed_x = _scatter_rows(x, perm_r, r_pad + TILE)

    wih_t = W_ih.T
    whh_t = W_hh.T
    bih2 = b_ih.reshape(1, 3 * h_dim)
    bhh2 = b_hh.reshape(1, 3 * h_dim)
    h0 = hidden_states[0]
    m0col = jnp.broadcast_to(
        ms[0].astype(jnp.float32).reshape(n, 1), (n, 128))

    packed_out = _packed_scan(packed_x, wih_t, whh_t, bih2, bhh2, h0,
                              m0col, meta, lanes16)

    out = _gather_rows(packed_out, perm_r, r_tot)
    h_final = out.reshape(t, n, h_dim)[-1][None]
    return out, h_final
